# baseline (speedup 1.0000x reference)
import jax
import jax.numpy as jnp
from jax import lax
from jax.experimental import pallas as pl
from jax.experimental.pallas import tpu as pltpu
from jax.experimental.pallas import tpu_sc as plsc

NCORE = 2
NSUB = 16
NW = NCORE * NSUB
LANES = 16
D = 64
W = 2 * D
KNEG = 8
MARGIN = 1.0
NPOS = 4096
TOTAL = 36864
N_ENT = 1000000

EB = 1024
EBITS = 10
NSB_FULL = N_ENT // EB
TAIL = N_ENT - NSB_FULL * EB
RAGGED = N_ENT % 128
ALIGNED_TAIL = TAIL - RAGGED
SB_ITERS = (NSB_FULL + NW) // NW
CB = 24
NBK = SB_ITERS
BKW = LANES * CB
DNCAP = LANES * CB + LANES
IDXC = 4096
NSCAN = TOTAL // IDXC
G_ROWS = 2 * TOTAL + 128
CHUNK = 128


def _extract_body(h_hbm, t_hbm, entT_hbm, tail_hbm, g_hbm,
                  idxbuf, idxbuf2, me, ms, cnts, dn_e, dn_s,
                  blk, tailbuf, extbuf, sidx,
                  sem, sem2, semi0, semi1):
    cid = lax.axis_index("c")
    sid = lax.axis_index("s")
    wid = sid * NCORE + cid
    lane = lax.iota(jnp.int32, LANES)

    def reset_sidx():
        for jg in range(CHUNK // LANES):
            plsc.store_scatter(sidx, [jg * LANES + lane],
                               2 * TOTAL + jg * LANES + lane)

    def scan_chunk(slot0, pass_base, ibuf):
        def v_body(v, carry):
            ev = ibuf[pl.ds(v * LANES, LANES)]
            m = ((ev >> EBITS) & (NW - 1)) == wid
            caddr = (ev >> (EBITS + 5)) * LANES + lane
            cur = plsc.load_gather(cnts, [caddr])
            keep = m & (cur >= pass_base) & (cur < pass_base + CB)
            addr = (ev >> (EBITS + 5)) * BKW + lane * CB + (cur - pass_base)
            plsc.store_scatter(me, [addr], ev, mask=keep)
            plsc.store_scatter(ms, [addr], slot0 + v * LANES + lane, mask=keep)
            plsc.store_scatter(cnts, [caddr], cur + 1, mask=m)
            return carry

        lax.fori_loop(0, IDXC // LANES, v_body, jnp.int32(0), unroll=8)

    def do_scan(pass_base):
        def z_body(b, carry):
            plsc.store_scatter(cnts, [b * LANES + lane],
                               jnp.zeros((LANES,), jnp.int32))
            return carry

        lax.fori_loop(0, NBK, z_body, jnp.int32(0))
        chunks = [(part, ic) for part, _ in ((0, h_hbm), (1, t_hbm))
                  for ic in range(NSCAN)]
        srcs = (h_hbm, t_hbm)
        ibufs = (idxbuf, idxbuf2)
        sems = (semi0, semi1)
        cp = pltpu.async_copy(srcs[0].at[pl.ds(0, IDXC)], ibufs[0], sems[0])
        for k, (part, ic) in enumerate(chunks):
            cp.wait()
            if k + 1 < len(chunks):
                npart, nic = chunks[k + 1]
                cp = pltpu.async_copy(
                    srcs[npart].at[pl.ds(nic * IDXC, IDXC)],
                    ibufs[(k + 1) % 2], sems[(k + 1) % 2])
            scan_chunk(part * TOTAL + ic * IDXC, pass_base, ibufs[k % 2])
        maxc = jnp.zeros((LANES,), jnp.int32)

        def m_body(b, maxc):
            return jnp.maximum(maxc, plsc.load_gather(cnts, [b * LANES + lane]))

        maxc = lax.fori_loop(0, NBK, m_body, maxc)
        return jnp.max(maxc)

    def extract_subblock(i, e_base, pass_base, pending, blk_copy=None):
        with jax.named_scope("p1_compact"):
            cnt16 = plsc.load_gather(cnts, [i * LANES + lane])
            fperc = jnp.clip(cnt16 - pass_base, 0, CB)
            epre = plsc.cumsum(fperc) - fperc
            dcnt = jnp.sum(fperc)
            for r in range(LANES):
                fr = fperc[r]
                pr = epre[r]

                def cp_body(v, c, r=r, fr=fr, pr=pr):
                    j = v * LANES + lane
                    ok = j < fr
                    ev = plsc.load_gather(me, [i * BKW + r * CB + j])
                    sv = plsc.load_gather(ms, [i * BKW + r * CB + j])
                    plsc.store_scatter(dn_e, [pr + j], ev, mask=ok)
                    plsc.store_scatter(dn_s, [pr + j], sv, mask=ok)
                    return c

                lax.fori_loop(0, (fr + LANES - 1) // LANES, cp_body,
                              jnp.int32(0))

        if blk_copy is not None:
            blk_copy.wait()

        def g_body(g, carry):
            @pl.when((g == 0) & (pending > 0))
            def _():
                pltpu.make_async_copy(g_hbm.at[pl.ds(0, CHUNK)], extbuf,
                                      sem).wait()
                reset_sidx()

            idx = g * LANES + lane
            valid = idx < dcnt
            e16 = plsc.load_gather(dn_e, [idx])
            s16 = plsc.load_gather(dn_s, [idx])
            col = jnp.where(valid, e16 - e_base, 0)
            grow = (g % 8) * LANES
            slot = jnp.where(valid, s16, 2 * TOTAL + grow + lane)
            plsc.store_scatter(sidx, [grow + lane], slot)

            for l in range(LANES):
                cl = col[l]
                for c in range(D // LANES):
                    dvec = jnp.full((LANES,), c * LANES, jnp.int32) + lane
                    v = plsc.load_gather(blk, [dvec, jnp.full((LANES,), 0,
                                                              jnp.int32) + cl])
                    extbuf[grow + l, pl.ds(c * LANES, LANES)] = v

            is_last = (g + 1) * LANES >= dcnt

            @pl.when((g % 8 == 7) & jnp.logical_not(is_last))
            def _():
                pltpu.async_copy(extbuf, g_hbm.at[sidx], sem).wait()
                reset_sidx()

            @pl.when(is_last)
            def _():
                pltpu.async_copy(extbuf, g_hbm.at[sidx], sem)

            return carry

        nb = (dcnt + LANES - 1) // LANES
        with jax.named_scope("p1_build"):
            lax.fori_loop(0, nb, g_body, jnp.int32(0))
        return jnp.where(dcnt > 0, jnp.int32(1), pending)

    def do_extract(pass_base, pending):
        def sb_body(i, pending):
            sb = wid + i * NW

            def full_case(pending):
                with jax.named_scope("p1_sbdma"):
                    cp = pltpu.async_copy(entT_hbm.at[:, pl.ds(sb * EB, EB)],
                                          blk, sem2)
                return extract_subblock(i, sb * EB, pass_base, pending, cp)

            def tail_case(pending):
                pltpu.sync_copy(entT_hbm.at[:, pl.ds(NSB_FULL * EB,
                                                     ALIGNED_TAIL)],
                                blk.at[:, pl.ds(0, ALIGNED_TAIL)])
                pltpu.sync_copy(tail_hbm, tailbuf)

                def tr_body(dd, carry):
                    dfull = jnp.full((LANES,), dd, jnp.int32)
                    for jg in range(RAGGED // LANES):
                        j16 = jnp.full((LANES,), jg * LANES, jnp.int32) + lane
                        v = plsc.load_gather(tailbuf, [j16, dfull])
                        plsc.store_scatter(blk, [dfull, ALIGNED_TAIL + j16], v)
                    return carry

                lax.fori_loop(0, D, tr_body, jnp.int32(0))
                return extract_subblock(i, NSB_FULL * EB, pass_base, pending)

            return lax.cond(sb < NSB_FULL, full_case,
                            lambda p: lax.cond(sb == NSB_FULL, tail_case,
                                               lambda q: q, p),
                            pending)

        return lax.fori_loop(0, SB_ITERS, sb_body, pending)

    reset_sidx()

    def p_cond(state):
        pass_base, maxtot, pending = state
        return (pass_base == 0) | (pass_base < maxtot)

    def p_body(state):
        pass_base, _, pending = state
        with jax.named_scope("p1_scan"):
            maxtot = do_scan(pass_base)
        pending = do_extract(pass_base, pending)
        return pass_base + CB, maxtot, pending

    _, _, pending = lax.while_loop(
        p_cond, p_body, (jnp.int32(0), jnp.int32(0), jnp.int32(0)))

    @pl.when(pending > 0)
    def _():
        pltpu.make_async_copy(g_hbm.at[pl.ds(0, CHUNK)], extbuf, sem).wait()


def _score_body(r_hbm, g_hbm, rel_hbm, out_hbm,
                idx_r0, pidx_r0, rows_h0, rows_t0, rows_r0,
                idx_r1, pidx_r1, rows_h1, rows_t1, rows_r1,
                scores_p, scores_n, loss_buf, sem0, sem1):
    cid = lax.axis_index("c")
    sid = lax.axis_index("s")
    wid = sid * NCORE + cid
    lane = lax.iota(jnp.int32, LANES)

    bufs = ((idx_r0, pidx_r0, rows_h0, rows_t0, rows_r0, sem0),
            (idx_r1, pidx_r1, rows_h1, rows_t1, rows_r1, sem1))

    def issue(base, buf):
        idx_r, _, rows_h, rows_t, _, sem = buf
        c1 = pltpu.async_copy(g_hbm.at[pl.ds(base, CHUNK)], rows_h, sem)
        c2 = pltpu.async_copy(g_hbm.at[pl.ds(TOTAL + base, CHUNK)], rows_t,
                              sem)
        c3 = pltpu.async_copy(r_hbm.at[pl.ds(base, CHUNK)], idx_r, sem)
        return c1, c2, c3

    def finish(buf, cps):
        idx_r, pidx_r, _, _, rows_r, sem = buf
        with jax.named_scope("p2_wait"):
            for c in cps:
                c.wait()

        def pair_body(g, carry):
            row0 = g * LANES + lane
            plsc.store_scatter(pidx_r, [row0],
                               plsc.load_gather(idx_r, [row0]) >> 1)
            return carry

        with jax.named_scope("p2_rel"):
            lax.fori_loop(0, CHUNK // LANES, pair_body, jnp.int32(0))
            pltpu.async_copy(rel_hbm.at[pidx_r], rows_r, sem).wait()

    def score_chunk(buf, scores_ref):
        idx_r, _, rows_h, rows_t, rows_r, _ = buf

        def g_body(g, carry):
            rbv = (idx_r[pl.ds(g * LANES, LANES)] & 1) * D
            sv = jnp.zeros((LANES,), jnp.float32)
            for l in range(LANES):
                t = g * LANES + l
                rb = rbv[l]
                acc = jnp.zeros((LANES,), jnp.float32)
                for c in range(D // LANES):
                    hv = rows_h[t, pl.ds(c * LANES, LANES)]
                    rv = rows_r[t, pl.ds(rb + c * LANES, LANES)]
                    tv = rows_t[t, pl.ds(c * LANES, LANES)]
                    acc = acc + jnp.abs(hv + rv - tv)
                sv = jnp.where(lane == l, jnp.sum(acc), sv)
            plsc.store_scatter(scores_ref, [g * LANES + lane], sv)
            return carry

        with jax.named_scope("p2_score"):
            lax.fori_loop(0, CHUNK // LANES, g_body, jnp.int32(0))

    neg0 = NPOS + wid * (CHUNK * KNEG)
    bases = [wid * CHUNK] + [neg0 + j * CHUNK for j in range(KNEG)]

    cps = issue(bases[0], bufs[0])
    finish(bufs[0], cps)
    loss_acc = jnp.zeros((LANES,), jnp.float32)
    for k, base in enumerate(bases):
        cur = bufs[k % 2]
        nxt = bufs[(k + 1) % 2]
        if k + 1 < len(bases):
            nxt_cps = issue(bases[k + 1], nxt)
        if k == 0:
            score_chunk(cur, scores_p)
        else:
            score_chunk(cur, scores_n)
            nacc = jnp.zeros((LANES,), jnp.float32)
            for kk in range(KNEG):
                nacc = nacc + plsc.load_gather(scores_n, [lane * KNEG + kk])
            p = scores_p[pl.ds((k - 1) * LANES, LANES)]
            loss_acc = loss_acc + jnp.maximum(
                p - nacc * (1.0 / KNEG) + MARGIN, 0.0)
        if k + 1 < len(bases):
            finish(nxt, nxt_cps)

    loss_buf[...] = loss_acc
    pltpu.sync_copy(loss_buf, out_hbm.at[pl.ds(wid * LANES, LANES)])


@jax.jit
def kernel(batch_h, batch_t, batch_r, batch_size, n_negative,
           ent_embeddings, rel_embeddings):
    del batch_size, n_negative
    entT = ent_embeddings.T
    rel2 = rel_embeddings.reshape(rel_embeddings.shape[0] // 2, W)
    mesh = plsc.VectorSubcoreMesh(core_axis_name="c", subcore_axis_name="s",
                                  num_cores=NCORE, num_subcores=NSUB)
    params = pltpu.CompilerParams(needs_layout_passes=False,
                                  use_tc_tiling_on_sc=True)

    kern1 = pl.kernel(
        _extract_body,
        out_type=jax.ShapeDtypeStruct((G_ROWS, W), jnp.float32),
        mesh=mesh,
        compiler_params=params,
        scratch_types=[
            pltpu.VMEM((IDXC,), jnp.int32),
            pltpu.VMEM((IDXC,), jnp.int32),
            pltpu.VMEM((NBK * BKW + LANES,), jnp.int32),
            pltpu.VMEM((NBK * BKW + LANES,), jnp.int32),
            pltpu.VMEM((NBK * LANES,), jnp.int32),
            pltpu.VMEM((DNCAP,), jnp.int32),
            pltpu.VMEM((DNCAP,), jnp.int32),
            pltpu.VMEM((D, EB), jnp.float32),
            pltpu.VMEM((RAGGED, D), jnp.float32),
            pltpu.VMEM((CHUNK, W), jnp.float32),
            pltpu.VMEM((CHUNK,), jnp.int32),
            pltpu.SemaphoreType.DMA,
            pltpu.SemaphoreType.DMA,
            pltpu.SemaphoreType.DMA,
            pltpu.SemaphoreType.DMA,
        ],
    )
    tail_rows = lax.slice(ent_embeddings, (N_ENT - RAGGED, 0), (N_ENT, D))
    g = kern1(batch_h, batch_t, entT, tail_rows)

    kern2 = pl.kernel(
        _score_body,
        out_type=jax.ShapeDtypeStruct((NW * LANES,), jnp.float32),
        mesh=mesh,
        compiler_params=params,
        scratch_types=[
            pltpu.VMEM((CHUNK,), jnp.int32),
            pltpu.VMEM((CHUNK,), jnp.int32),
            pltpu.VMEM((CHUNK, W), jnp.float32),
            pltpu.VMEM((CHUNK, W), jnp.float32),
            pltpu.VMEM((CHUNK, W), jnp.float32),
            pltpu.VMEM((CHUNK,), jnp.int32),
            pltpu.VMEM((CHUNK,), jnp.int32),
            pltpu.VMEM((CHUNK, W), jnp.float32),
            pltpu.VMEM((CHUNK, W), jnp.float32),
            pltpu.VMEM((CHUNK, W), jnp.float32),
            pltpu.VMEM((CHUNK,), jnp.float32),
            pltpu.VMEM((CHUNK,), jnp.float32),
            pltpu.VMEM((LANES,), jnp.float32),
            pltpu.SemaphoreType.DMA,
            pltpu.SemaphoreType.DMA,
        ],
    )
    partials = kern2(batch_r, g, rel2)
    return jnp.sum(partials)


# speedup vs baseline: 1.0393x; 1.0393x over previous
import jax
import jax.numpy as jnp
from jax import lax
from jax.experimental import pallas as pl
from jax.experimental.pallas import tpu as pltpu
from jax.experimental.pallas import tpu_sc as plsc

NCORE = 2
NSUB = 16
NW = NCORE * NSUB
LANES = 16
D = 64
W = 2 * D
KNEG = 8
MARGIN = 1.0
NPOS = 4096
TOTAL = 36864
N_ENT = 1000000

EB = 1024
EBITS = 10
NSB_FULL = N_ENT // EB
TAIL = N_ENT - NSB_FULL * EB
RAGGED = N_ENT % 128
ALIGNED_TAIL = TAIL - RAGGED
SB_ITERS = (NSB_FULL + NW) // NW
CB = 24
NBK = SB_ITERS
BKW = LANES * CB
DNCAP = LANES * CB + LANES
IDXC = 4096
NSCAN = TOTAL // IDXC
G_ROWS = 2 * TOTAL + 128
CHUNK = 128


def _extract_body(h_hbm, t_hbm, entT_hbm, tail_hbm, g_hbm,
                  idxbuf, idxbuf2, me, ms, cnts, dn_e, dn_s,
                  blk, tailbuf, extbuf, sidx,
                  sem, sem2, semi0, semi1):
    cid = lax.axis_index("c")
    sid = lax.axis_index("s")
    wid = sid * NCORE + cid
    lane = lax.iota(jnp.int32, LANES)

    def reset_sidx():
        for jg in range(CHUNK // LANES):
            plsc.store_scatter(sidx, [jg * LANES + lane],
                               2 * TOTAL + jg * LANES + lane)

    def scan_chunk(slot0, pass_base, ibuf):
        def v_body(v, carry):
            ev = ibuf[pl.ds(v * LANES, LANES)]
            m = ((ev >> EBITS) & (NW - 1)) == wid
            caddr = (ev >> (EBITS + 5)) * LANES + lane
            cur = plsc.load_gather(cnts, [caddr])
            keep = m & (cur >= pass_base) & (cur < pass_base + CB)
            addr = (ev >> (EBITS + 5)) * BKW + lane * CB + (cur - pass_base)
            plsc.store_scatter(me, [addr], ev, mask=keep)
            plsc.store_scatter(ms, [addr], slot0 + v * LANES + lane, mask=keep)
            plsc.store_scatter(cnts, [caddr], cur + 1, mask=m)
            return carry

        lax.fori_loop(0, IDXC // LANES, v_body, jnp.int32(0), unroll=8)

    def do_scan(pass_base):
        def z_body(b, carry):
            plsc.store_scatter(cnts, [b * LANES + lane],
                               jnp.zeros((LANES,), jnp.int32))
            return carry

        lax.fori_loop(0, NBK, z_body, jnp.int32(0))
        chunks = [(part, ic) for part, _ in ((0, h_hbm), (1, t_hbm))
                  for ic in range(NSCAN)]
        srcs = (h_hbm, t_hbm)
        ibufs = (idxbuf, idxbuf2)
        sems = (semi0, semi1)
        cp = pltpu.async_copy(srcs[0].at[pl.ds(0, IDXC)], ibufs[0], sems[0])
        for k, (part, ic) in enumerate(chunks):
            cp.wait()
            if k + 1 < len(chunks):
                npart, nic = chunks[k + 1]
                cp = pltpu.async_copy(
                    srcs[npart].at[pl.ds(nic * IDXC, IDXC)],
                    ibufs[(k + 1) % 2], sems[(k + 1) % 2])
            scan_chunk(part * TOTAL + ic * IDXC, pass_base, ibufs[k % 2])
        maxc = jnp.zeros((LANES,), jnp.int32)

        def m_body(b, maxc):
            return jnp.maximum(maxc, plsc.load_gather(cnts, [b * LANES + lane]))

        maxc = lax.fori_loop(0, NBK, m_body, maxc)
        return jnp.max(maxc)

    def extract_subblock(i, e_base, pass_base, pending, blk_copy=None):
        with jax.named_scope("p1_compact"):
            cnt16 = plsc.load_gather(cnts, [i * LANES + lane])
            fperc = jnp.clip(cnt16 - pass_base, 0, CB)
            epre = plsc.cumsum(fperc) - fperc
            dcnt = jnp.sum(fperc)
            for r in range(LANES):
                fr = fperc[r]
                pr = epre[r]

                def cp_body(v, c, r=r, fr=fr, pr=pr):
                    j = v * LANES + lane
                    ok = j < fr
                    ev = plsc.load_gather(me, [i * BKW + r * CB + j])
                    sv = plsc.load_gather(ms, [i * BKW + r * CB + j])
                    plsc.store_scatter(dn_e, [pr + j], ev, mask=ok)
                    plsc.store_scatter(dn_s, [pr + j], sv, mask=ok)
                    return c

                lax.fori_loop(0, (fr + LANES - 1) // LANES, cp_body,
                              jnp.int32(0))

        if blk_copy is not None:
            blk_copy.wait()

        def g_body(g, carry):
            @pl.when((g == 0) & (pending > 0))
            def _():
                pltpu.make_async_copy(g_hbm.at[pl.ds(0, CHUNK)], extbuf,
                                      sem).wait()
                reset_sidx()

            idx = g * LANES + lane
            valid = idx < dcnt
            e16 = plsc.load_gather(dn_e, [idx])
            s16 = plsc.load_gather(dn_s, [idx])
            col = jnp.where(valid, e16 - e_base, 0)
            grow = (g % 8) * LANES
            slot = jnp.where(valid, s16, 2 * TOTAL + grow + lane)
            plsc.store_scatter(sidx, [grow + lane], slot)
            rowv = jnp.full((LANES,), 0, jnp.int32) + grow + lane

            def d_body(dd, dfull):
                v = plsc.load_gather(blk, [dfull, col])
                plsc.store_scatter(extbuf, [rowv, dfull], v)
                return dfull + 1

            lax.fori_loop(0, D, d_body, jnp.zeros((LANES,), jnp.int32),
                          unroll=8)

            is_last = (g + 1) * LANES >= dcnt

            @pl.when((g % 8 == 7) & jnp.logical_not(is_last))
            def _():
                pltpu.async_copy(extbuf, g_hbm.at[sidx], sem).wait()
                reset_sidx()

            @pl.when(is_last)
            def _():
                pltpu.async_copy(extbuf, g_hbm.at[sidx], sem)

            return carry

        nb = (dcnt + LANES - 1) // LANES
        with jax.named_scope("p1_build"):
            lax.fori_loop(0, nb, g_body, jnp.int32(0))
        return jnp.where(dcnt > 0, jnp.int32(1), pending)

    def do_extract(pass_base, pending):
        def sb_body(i, pending):
            sb = wid + i * NW

            def full_case(pending):
                with jax.named_scope("p1_sbdma"):
                    cp = pltpu.async_copy(entT_hbm.at[:, pl.ds(sb * EB, EB)],
                                          blk, sem2)
                return extract_subblock(i, sb * EB, pass_base, pending, cp)

            def tail_case(pending):
                pltpu.sync_copy(entT_hbm.at[:, pl.ds(NSB_FULL * EB,
                                                     ALIGNED_TAIL)],
                                blk.at[:, pl.ds(0, ALIGNED_TAIL)])
                pltpu.sync_copy(tail_hbm, tailbuf)

                def tr_body(dd, carry):
                    dfull = jnp.full((LANES,), dd, jnp.int32)
                    for jg in range(RAGGED // LANES):
                        j16 = jnp.full((LANES,), jg * LANES, jnp.int32) + lane
                        v = plsc.load_gather(tailbuf, [j16, dfull])
                        plsc.store_scatter(blk, [dfull, ALIGNED_TAIL + j16], v)
                    return carry

                lax.fori_loop(0, D, tr_body, jnp.int32(0))
                return extract_subblock(i, NSB_FULL * EB, pass_base, pending)

            return lax.cond(sb < NSB_FULL, full_case,
                            lambda p: lax.cond(sb == NSB_FULL, tail_case,
                                               lambda q: q, p),
                            pending)

        return lax.fori_loop(0, SB_ITERS, sb_body, pending)

    reset_sidx()

    def p_cond(state):
        pass_base, maxtot, pending = state
        return (pass_base == 0) | (pass_base < maxtot)

    def p_body(state):
        pass_base, _, pending = state
        with jax.named_scope("p1_scan"):
            maxtot = do_scan(pass_base)
        pending = do_extract(pass_base, pending)
        return pass_base + CB, maxtot, pending

    _, _, pending = lax.while_loop(
        p_cond, p_body, (jnp.int32(0), jnp.int32(0), jnp.int32(0)))

    @pl.when(pending > 0)
    def _():
        pltpu.make_async_copy(g_hbm.at[pl.ds(0, CHUNK)], extbuf, sem).wait()


def _score_body(r_hbm, g_hbm, rel_hbm, out_hbm,
                idx_r0, pidx_r0, rows_h0, rows_t0, rows_r0,
                idx_r1, pidx_r1, rows_h1, rows_t1, rows_r1,
                scores_p, scores_n, loss_buf, sem0, sem1):
    cid = lax.axis_index("c")
    sid = lax.axis_index("s")
    wid = sid * NCORE + cid
    lane = lax.iota(jnp.int32, LANES)

    bufs = ((idx_r0, pidx_r0, rows_h0, rows_t0, rows_r0, sem0),
            (idx_r1, pidx_r1, rows_h1, rows_t1, rows_r1, sem1))

    def issue(base, buf):
        idx_r, _, rows_h, rows_t, _, sem = buf
        c1 = pltpu.async_copy(g_hbm.at[pl.ds(base, CHUNK)], rows_h, sem)
        c2 = pltpu.async_copy(g_hbm.at[pl.ds(TOTAL + base, CHUNK)], rows_t,
                              sem)
        c3 = pltpu.async_copy(r_hbm.at[pl.ds(base, CHUNK)], idx_r, sem)
        return c1, c2, c3

    def finish(buf, cps):
        idx_r, pidx_r, _, _, rows_r, sem = buf
        with jax.named_scope("p2_wait"):
            for c in cps:
                c.wait()

        def pair_body(g, carry):
            row0 = g * LANES + lane
            plsc.store_scatter(pidx_r, [row0],
                               plsc.load_gather(idx_r, [row0]) >> 1)
            return carry

        with jax.named_scope("p2_rel"):
            lax.fori_loop(0, CHUNK // LANES, pair_body, jnp.int32(0))
            pltpu.async_copy(rel_hbm.at[pidx_r], rows_r, sem).wait()

    def score_chunk(buf, scores_ref):
        idx_r, _, rows_h, rows_t, rows_r, _ = buf

        def g_body(g, carry):
            rbv = (idx_r[pl.ds(g * LANES, LANES)] & 1) * D
            sv = jnp.zeros((LANES,), jnp.float32)
            for l in range(LANES):
                t = g * LANES + l
                rb = rbv[l]
                acc = jnp.zeros((LANES,), jnp.float32)
                for c in range(D // LANES):
                    hv = rows_h[t, pl.ds(c * LANES, LANES)]
                    rv = rows_r[t, pl.ds(rb + c * LANES, LANES)]
                    tv = rows_t[t, pl.ds(c * LANES, LANES)]
                    acc = acc + jnp.abs(hv + rv - tv)
                sv = jnp.where(lane == l, jnp.sum(acc), sv)
            plsc.store_scatter(scores_ref, [g * LANES + lane], sv)
            return carry

        with jax.named_scope("p2_score"):
            lax.fori_loop(0, CHUNK // LANES, g_body, jnp.int32(0))

    neg0 = NPOS + wid * (CHUNK * KNEG)
    bases = [wid * CHUNK] + [neg0 + j * CHUNK for j in range(KNEG)]

    cps = issue(bases[0], bufs[0])
    finish(bufs[0], cps)
    loss_acc = jnp.zeros((LANES,), jnp.float32)
    for k, base in enumerate(bases):
        cur = bufs[k % 2]
        nxt = bufs[(k + 1) % 2]
        if k + 1 < len(bases):
            nxt_cps = issue(bases[k + 1], nxt)
        if k == 0:
            score_chunk(cur, scores_p)
        else:
            score_chunk(cur, scores_n)
            nacc = jnp.zeros((LANES,), jnp.float32)
            for kk in range(KNEG):
                nacc = nacc + plsc.load_gather(scores_n, [lane * KNEG + kk])
            p = scores_p[pl.ds((k - 1) * LANES, LANES)]
            loss_acc = loss_acc + jnp.maximum(
                p - nacc * (1.0 / KNEG) + MARGIN, 0.0)
        if k + 1 < len(bases):
            finish(nxt, nxt_cps)

    loss_buf[...] = loss_acc
    pltpu.sync_copy(loss_buf, out_hbm.at[pl.ds(wid * LANES, LANES)])


@jax.jit
def kernel(batch_h, batch_t, batch_r, batch_size, n_negative,
           ent_embeddings, rel_embeddings):
    del batch_size, n_negative
    entT = ent_embeddings.T
    rel2 = rel_embeddings.reshape(rel_embeddings.shape[0] // 2, W)
    mesh = plsc.VectorSubcoreMesh(core_axis_name="c", subcore_axis_name="s",
                                  num_cores=NCORE, num_subcores=NSUB)
    params = pltpu.CompilerParams(needs_layout_passes=False,
                                  use_tc_tiling_on_sc=True)

    kern1 = pl.kernel(
        _extract_body,
        out_type=jax.ShapeDtypeStruct((G_ROWS, W), jnp.float32),
        mesh=mesh,
        compiler_params=params,
        scratch_types=[
            pltpu.VMEM((IDXC,), jnp.int32),
            pltpu.VMEM((IDXC,), jnp.int32),
            pltpu.VMEM((NBK * BKW + LANES,), jnp.int32),
            pltpu.VMEM((NBK * BKW + LANES,), jnp.int32),
            pltpu.VMEM((NBK * LANES,), jnp.int32),
            pltpu.VMEM((DNCAP,), jnp.int32),
            pltpu.VMEM((DNCAP,), jnp.int32),
            pltpu.VMEM((D, EB), jnp.float32),
            pltpu.VMEM((RAGGED, D), jnp.float32),
            pltpu.VMEM((CHUNK, W), jnp.float32),
            pltpu.VMEM((CHUNK,), jnp.int32),
            pltpu.SemaphoreType.DMA,
            pltpu.SemaphoreType.DMA,
            pltpu.SemaphoreType.DMA,
            pltpu.SemaphoreType.DMA,
        ],
    )
    tail_rows = lax.slice(ent_embeddings, (N_ENT - RAGGED, 0), (N_ENT, D))
    g = kern1(batch_h, batch_t, entT, tail_rows)

    kern2 = pl.kernel(
        _score_body,
        out_type=jax.ShapeDtypeStruct((NW * LANES,), jnp.float32),
        mesh=mesh,
        compiler_params=params,
        scratch_types=[
            pltpu.VMEM((CHUNK,), jnp.int32),
            pltpu.VMEM((CHUNK,), jnp.int32),
            pltpu.VMEM((CHUNK, W), jnp.float32),
            pltpu.VMEM((CHUNK, W), jnp.float32),
            pltpu.VMEM((CHUNK, W), jnp.float32),
            pltpu.VMEM((CHUNK,), jnp.int32),
            pltpu.VMEM((CHUNK,), jnp.int32),
            pltpu.VMEM((CHUNK, W), jnp.float32),
            pltpu.VMEM((CHUNK, W), jnp.float32),
            pltpu.VMEM((CHUNK, W), jnp.float32),
            pltpu.VMEM((CHUNK,), jnp.float32),
            pltpu.VMEM((CHUNK,), jnp.float32),
            pltpu.VMEM((LANES,), jnp.float32),
            pltpu.SemaphoreType.DMA,
            pltpu.SemaphoreType.DMA,
        ],
    )
    partials = kern2(batch_r, g, rel2)
    return jnp.sum(partials)
